# baseline (speedup 1.0000x reference)
import functools

import jax
import jax.numpy as jnp
from jax import lax
from jax.experimental import pallas as pl
from jax.experimental.pallas import tpu as pltpu
from jax.experimental.pallas import tpu_sc as plsc

MAXREL = 128
D = 32
T = 2 * MAXREL + 1
NC = 2
NS = 16
NW = NC * NS
LANES = 16


def _make_sc_kernel(L: int):
    R = L // NW
    JT = L // 128
    NQ = 4
    BLK = L // NQ
    win_pad = L + BLK
    n_chunks = win_pad // LANES

    mesh = plsc.VectorSubcoreMesh(
        core_axis_name="c", subcore_axis_name="s",
        num_cores=NC, num_subcores=NS)

    @functools.partial(
        pl.kernel,
        out_type=jax.ShapeDtypeStruct((L, D // 8, JT, 8, 128), jnp.float32),
        mesh=mesh,
        scratch_types=[
            pltpu.VMEM((T * D,), jnp.float32),
            pltpu.VMEM((D // 8, 8, win_pad), jnp.float32),
            pltpu.VMEM((win_pad,), jnp.int32),
            pltpu.SemaphoreType.DMA,
        ],
        compiler_params=pltpu.CompilerParams(
            use_tc_tiling_on_sc=False, needs_layout_passes=False),
    )
    def retrieve(table_hbm, out_hbm, table_v, sbt, idxb, sem):
        c = lax.axis_index("c")
        s = lax.axis_index("s")
        wid = c * NS + s
        p = wid & 7
        q = wid >> 3
        ibase = q * BLK + p

        pltpu.sync_copy(table_hbm, table_v)

        off = MAXREL - ibase - 8 * (R - 1)
        iota = lax.iota(jnp.int32, LANES)

        def build_idx(k, carry):
            c0 = k * LANES
            idxb[pl.ds(c0, LANES)] = jnp.clip(c0 + iota + off, 0, T - 1) * D
            return carry

        lax.fori_loop(0, n_chunks, build_idx, 0)

        lead_chunks = (8 * R + 2 * 128) // LANES

        def build_span(lo, hi):
            def body(n, carry):
                d = n & (D - 1)
                k = lo + (n >> 5)
                c0 = k * LANES
                vals = plsc.load_gather(table_v, [idxb[pl.ds(c0, LANES)] + d])
                sbt[d >> 3, d & 7, pl.ds(c0, LANES)] = vals
                return carry

            lax.fori_loop(0, D * (hi - lo), body, 0)

        def issue_tile(jt):
            def body(t, carry):
                i = ibase + 8 * t
                cb = 8 * (R - 1 - t)
                pltpu.async_copy(
                    sbt.at[:, :, pl.ds(cb + 128 * jt, 128)],
                    out_hbm.at[i, :, jt],
                    sem)
                return carry

            lax.fori_loop(0, R, body, 0)

        build_span(0, lead_chunks)
        for jt in range(JT):
            issue_tile(jt)
            nxt = lead_chunks + (jt + 1) * 8
            if nxt <= n_chunks:
                build_span(nxt - 8, nxt)

        def drain(n, carry):
            pltpu.make_async_copy(
                sbt.at[:, :, pl.ds(0, 128)],
                out_hbm.at[ibase, :, 0],
                sem).wait()
            return carry

        lax.fori_loop(0, R * JT, drain, 0)

    return retrieve


def kernel(x, rel_embeds):
    L = x.shape[1]
    out5 = _make_sc_kernel(L)(rel_embeds.reshape(-1))
    return out5.transpose(0, 2, 4, 1, 3).reshape(L, L, D)


# speedup vs baseline: 1.0655x; 1.0655x over previous
import functools

import jax
import jax.numpy as jnp
from jax import lax
from jax.experimental import pallas as pl
from jax.experimental.pallas import tpu as pltpu
from jax.experimental.pallas import tpu_sc as plsc

MAXREL = 128
D = 32
T = 2 * MAXREL + 1
NC = 2
NS = 16
NW = NC * NS
LANES = 16


def _make_sc_kernel(L: int):
    R = L // NW
    JT = L // 128
    NQ = 4
    BLK = L // NQ
    win_pad = L + BLK
    n_chunks = win_pad // LANES

    mesh = plsc.VectorSubcoreMesh(
        core_axis_name="c", subcore_axis_name="s",
        num_cores=NC, num_subcores=NS)

    @functools.partial(
        pl.kernel,
        out_type=jax.ShapeDtypeStruct((L, D // 8, JT, 8, 128), jnp.float32),
        mesh=mesh,
        scratch_types=[
            pltpu.VMEM((T * D,), jnp.float32),
            pltpu.VMEM((D // 8, 8, win_pad), jnp.float32),
            pltpu.VMEM((win_pad,), jnp.int32),
            pltpu.SemaphoreType.DMA,
        ],
        compiler_params=pltpu.CompilerParams(
            use_tc_tiling_on_sc=False, needs_layout_passes=False),
    )
    def retrieve(table_hbm, out_hbm, table_v, sbt, idxb, sem):
        c = lax.axis_index("c")
        s = lax.axis_index("s")
        wid = c * NS + s
        p = wid & 7
        q = wid >> 3
        ibase = q * BLK + p

        pltpu.sync_copy(table_hbm, table_v)

        off = MAXREL - ibase - 8 * (R - 1)
        iota = lax.iota(jnp.int32, LANES)

        def build_idx(k, carry):
            c0 = k * LANES
            idxb[pl.ds(c0, LANES)] = jnp.clip(c0 + iota + off, 0, T - 1) * D
            return carry

        lax.fori_loop(0, n_chunks, build_idx, 0)

        for d in range(D):
            dt, ds = d >> 3, d & 7

            def build_chunk(k, carry, d=d, dt=dt, ds=ds):
                c0 = k * LANES
                vals = plsc.load_gather(table_v, [idxb[pl.ds(c0, LANES)] + d])
                sbt[dt, ds, pl.ds(c0, LANES)] = vals
                return carry

            lax.fori_loop(0, n_chunks, build_chunk, 0)

        def issue_row(t, carry):
            i = ibase + 8 * t
            cb = 8 * (R - 1 - t)
            for jt in range(JT):
                pltpu.async_copy(
                    sbt.at[:, :, pl.ds(cb + 128 * jt, 128)],
                    out_hbm.at[i, :, jt],
                    sem)
            return carry

        lax.fori_loop(0, R, issue_row, 0)

        def drain(n, carry):
            pltpu.make_async_copy(
                sbt.at[:, :, pl.ds(0, 128)],
                out_hbm.at[ibase, :, 0],
                sem).wait()
            return carry

        lax.fori_loop(0, R * JT, drain, 0)

    return retrieve


def kernel(x, rel_embeds):
    L = x.shape[1]
    out5 = _make_sc_kernel(L)(rel_embeds.reshape(-1))
    return out5.transpose(0, 2, 4, 1, 3).reshape(L, L, D)


# speedup vs baseline: 1.0902x; 1.0232x over previous
import functools

import jax
import jax.numpy as jnp
from jax import lax
from jax.experimental import pallas as pl
from jax.experimental.pallas import tpu as pltpu
from jax.experimental.pallas import tpu_sc as plsc

MAXREL = 128
D = 32
T = 2 * MAXREL + 1
NC = 2
NS = 16
NW = NC * NS
LANES = 16


def _make_sc_kernel(L: int):
    R = L // NW
    JT = L // 128
    NQ = 4
    BLK = L // NQ
    win_pad = L + BLK
    n_chunks = win_pad // LANES

    mesh = plsc.VectorSubcoreMesh(
        core_axis_name="c", subcore_axis_name="s",
        num_cores=NC, num_subcores=NS)

    @functools.partial(
        pl.kernel,
        out_type=jax.ShapeDtypeStruct((L, D // 8, JT, 8, 128), jnp.float32),
        mesh=mesh,
        scratch_types=[
            pltpu.VMEM((T * D,), jnp.float32),
            pltpu.VMEM((D // 8, 8, win_pad), jnp.float32),
            pltpu.SemaphoreType.DMA,
        ],
        compiler_params=pltpu.CompilerParams(
            use_tc_tiling_on_sc=False, needs_layout_passes=False),
    )
    def retrieve(table_hbm, out_hbm, table_v, sbt, sem):
        c = lax.axis_index("c")
        s = lax.axis_index("s")
        wid = c * NS + s
        p = wid & 7
        q = wid >> 3
        ibase = q * BLK + p

        pltpu.sync_copy(table_hbm, table_v)

        off = MAXREL - ibase - 8 * (R - 1)
        iota = lax.iota(jnp.int32, LANES)

        def build_span(lo, hi):
            for d in range(D):
                dt, ds = d >> 3, d & 7

                def build_chunk(k, carry, d=d, dt=dt, ds=ds):
                    c0 = k * LANES
                    src = jnp.clip(c0 + iota + off, 0, T - 1)
                    vals = plsc.load_gather(table_v, [src * D + d])
                    sbt[dt, ds, pl.ds(c0, LANES)] = vals
                    return carry

                lax.fori_loop(lo, hi, build_chunk, 0)

        def issue_tiles(jt_lo, jt_hi):
            def issue_row(t, carry):
                i = ibase + 8 * t
                cb = 8 * (R - 1 - t)
                for jt in range(jt_lo, jt_hi):
                    pltpu.async_copy(
                        sbt.at[:, :, pl.ds(cb + 128 * jt, 128)],
                        out_hbm.at[i, :, jt],
                        sem)
                return carry

            lax.fori_loop(0, R, issue_row, 0)

        build_span(0, n_chunks // 2)
        issue_tiles(0, 5)
        build_span(n_chunks // 2, n_chunks)
        issue_tiles(5, JT)

        def drain_row(t, carry):
            i = ibase + 8 * t
            cb = 8 * (R - 1 - t)
            for jt in range(JT):
                pltpu.make_async_copy(
                    sbt.at[:, :, pl.ds(cb + 128 * jt, 128)],
                    out_hbm.at[i, :, jt],
                    sem).wait()
            return carry

        lax.fori_loop(0, R, drain_row, 0)

    return retrieve


def kernel(x, rel_embeds):
    L = x.shape[1]
    out5 = _make_sc_kernel(L)(rel_embeds.reshape(-1))
    return out5.transpose(0, 2, 4, 1, 3).reshape(L, L, D)
